# baseline (speedup 1.0000x reference)
import jax
import jax.numpy as jnp
from jax import lax
from jax.experimental import pallas as pl
from jax.experimental.pallas import tpu as pltpu
from jax.experimental.pallas import tpu_sc as plsc

_BATCH = 16384
_FEAT = 64
_FPAD = 128
_LANES = 16
_NC = 2
_NS = 16
_NW = _NC * _NS
_BPW = _BATCH // _NW
_CHUNKS = 4
_RPC = _BPW // _CHUNKS


def _tec_body(feat_hbm, tgt_hbm, cent_hbm, out_hbm,
              idx_v, rows_v, feat_v, part_v, sem):
    c = lax.axis_index("c")
    s = lax.axis_index("s")
    wid = s * _NC + c
    base = wid * _BPW

    pltpu.sync_copy(tgt_hbm.at[pl.ds(base, _BPW)], idx_v)
    gathers = [
        pltpu.async_copy(cent_hbm.at[idx_v.at[pl.ds(j * _RPC, _RPC)]],
                         rows_v.at[pl.ds(j * _RPC, _RPC)], sem)
        for j in range(_CHUNKS)
    ]
    pltpu.sync_copy(feat_hbm.at[pl.ds(base, _BPW)], feat_v)
    for g in gathers:
        g.wait()

    zeros = jnp.zeros((_LANES,), jnp.float32)

    def body(r, accs):
        a0, a1, a2, a3 = accs
        d0 = feat_v[r, pl.ds(0, 16)] - rows_v[r, pl.ds(0, 16)]
        d1 = feat_v[r, pl.ds(16, 16)] - rows_v[r, pl.ds(16, 16)]
        d2 = feat_v[r, pl.ds(32, 16)] - rows_v[r, pl.ds(32, 16)]
        d3 = feat_v[r, pl.ds(48, 16)] - rows_v[r, pl.ds(48, 16)]
        return (a0 + d0 * d0, a1 + d1 * d1, a2 + d2 * d2, a3 + d3 * d3)

    a0, a1, a2, a3 = lax.fori_loop(0, _BPW, body, (zeros, zeros, zeros, zeros))
    part = ((a0 + a1) + (a2 + a3)) * (0.5 / _BATCH)
    part_v[...] = part
    pltpu.sync_copy(part_v, out_hbm.at[wid])


def _center_loss(features_pad, targets, centers):
    mesh = plsc.VectorSubcoreMesh(core_axis_name="c", subcore_axis_name="s")
    run = pl.kernel(
        _tec_body,
        mesh=mesh,
        out_type=jax.ShapeDtypeStruct((_NW, _LANES), jnp.float32),
        scratch_types=[
            pltpu.VMEM((_BPW,), jnp.int32),
            pltpu.VMEM((_BPW, _FEAT), jnp.float32),
            pltpu.VMEM((_BPW, _FPAD), jnp.float32),
            pltpu.VMEM((_LANES,), jnp.float32),
            pltpu.SemaphoreType.DMA,
        ],
        compiler_params=pltpu.CompilerParams(use_tc_tiling_on_sc=False),
    )
    parts = run(features_pad, targets, centers)
    return jnp.sum(parts)


def kernel(features, targets, centers):
    features_pad = jnp.pad(features, ((0, 0), (0, _FPAD - _FEAT)))
    return _center_loss(features_pad, targets.astype(jnp.int32), centers)


# speedup vs baseline: 1.8327x; 1.8327x over previous
import jax
import jax.numpy as jnp
from jax import lax
from jax.experimental import pallas as pl
from jax.experimental.pallas import tpu as pltpu
from jax.experimental.pallas import tpu_sc as plsc

_BATCH = 16384
_FEAT = 64
_CLASSES = 100000
_LANES = 16
_NC = 2
_NS = 16
_NW = _NC * _NS
_DPW = _FEAT // _NW
_TCHUNK = 4096


def _tec_body(ft_hbm, tgt_hbm, ct_hbm, out_hbm,
              crow_v, frow_v, tgt_v, part_v, sem):
    c = lax.axis_index("c")
    s = lax.axis_index("s")
    wid = s * _NC + c

    acc = jnp.zeros((_LANES,), jnp.float32)
    for j in range(_DPW):
        d = wid * _DPW + j
        pltpu.sync_copy(ct_hbm.at[d], crow_v)
        pltpu.sync_copy(ft_hbm.at[d], frow_v)
        for t in range(_BATCH // _TCHUNK):
            pltpu.sync_copy(tgt_hbm.at[pl.ds(t * _TCHUNK, _TCHUNK)], tgt_v)

            def step(k, a, _t=t):
                i0 = k * _LANES
                tv = tgt_v[pl.ds(i0, _LANES)]
                fv = frow_v[pl.ds(_t * _TCHUNK + i0, _LANES)]
                cv = plsc.load_gather(crow_v, [tv])
                dv = fv - cv
                return a + dv * dv

            acc = lax.fori_loop(0, _TCHUNK // _LANES, step, acc)

    part_v[...] = acc * (0.5 / _BATCH)
    pltpu.sync_copy(part_v, out_hbm.at[wid])


def _center_loss(features_t, targets, centers_t):
    mesh = plsc.VectorSubcoreMesh(core_axis_name="c", subcore_axis_name="s")
    run = pl.kernel(
        _tec_body,
        mesh=mesh,
        out_type=jax.ShapeDtypeStruct((_NW, _LANES), jnp.float32),
        scratch_types=[
            pltpu.VMEM((_CLASSES,), jnp.float32),
            pltpu.VMEM((_BATCH,), jnp.float32),
            pltpu.VMEM((_TCHUNK,), jnp.int32),
            pltpu.VMEM((_LANES,), jnp.float32),
            pltpu.SemaphoreType.DMA,
        ],
        compiler_params=pltpu.CompilerParams(needs_layout_passes=False),
    )
    parts = run(features_t, targets, centers_t)
    return jnp.sum(parts)


def kernel(features, targets, centers):
    return _center_loss(features.T, targets.astype(jnp.int32), centers.T)


# speedup vs baseline: 2.2896x; 1.2493x over previous
import jax
import jax.numpy as jnp
from jax import lax
from jax.experimental import pallas as pl
from jax.experimental.pallas import tpu as pltpu
from jax.experimental.pallas import tpu_sc as plsc

_BATCH = 16384
_FEAT = 64
_CLASSES = 100000
_LANES = 16
_NC = 2
_NS = 16
_NW = _NC * _NS
_DPW = _FEAT // _NW
_TCHUNK = 4096
_NTC = _BATCH // _TCHUNK
_UNROLL = 4


def _tec_body(ft_hbm, tgt_hbm, ct_hbm, out_hbm,
              crow_v, frow_v, tgt_v, part_v, sem, tsem):
    c = lax.axis_index("c")
    s = lax.axis_index("s")
    wid = s * _NC + c

    acc = (jnp.zeros((_LANES,), jnp.float32),) * _UNROLL
    for j in range(_DPW):
        d = wid * _DPW + j
        cp_c = pltpu.async_copy(ct_hbm.at[d], crow_v, sem)
        cp_f = pltpu.async_copy(ft_hbm.at[d], frow_v, sem)
        cp_t = pltpu.async_copy(tgt_hbm.at[pl.ds(0, _TCHUNK)], tgt_v.at[0],
                                tsem)
        cp_c.wait()
        cp_f.wait()
        for t in range(_NTC):
            cp_t.wait()
            if t + 1 < _NTC:
                cp_t = pltpu.async_copy(
                    tgt_hbm.at[pl.ds((t + 1) * _TCHUNK, _TCHUNK)],
                    tgt_v.at[(t + 1) % 2], tsem)

            def step(k, a, _t=t):
                i0 = k * (_LANES * _UNROLL)
                res = []
                for u in range(_UNROLL):
                    off = i0 + u * _LANES
                    tv = tgt_v[_t % 2, pl.ds(off, _LANES)]
                    fv = frow_v[pl.ds(_t * _TCHUNK + off, _LANES)]
                    cv = plsc.load_gather(crow_v, [tv])
                    dv = fv - cv
                    res.append(a[u] + dv * dv)
                return tuple(res)

            acc = lax.fori_loop(0, _TCHUNK // (_LANES * _UNROLL), step, acc)

    part = ((acc[0] + acc[1]) + (acc[2] + acc[3])) * (0.5 / _BATCH)
    part_v[...] = part
    pltpu.sync_copy(part_v, out_hbm.at[wid])


def _center_loss(features_t, targets, centers_t):
    mesh = plsc.VectorSubcoreMesh(core_axis_name="c", subcore_axis_name="s")
    run = pl.kernel(
        _tec_body,
        mesh=mesh,
        out_type=jax.ShapeDtypeStruct((_NW, _LANES), jnp.float32),
        scratch_types=[
            pltpu.VMEM((_CLASSES,), jnp.float32),
            pltpu.VMEM((_BATCH,), jnp.float32),
            pltpu.VMEM((2, _TCHUNK), jnp.int32),
            pltpu.VMEM((_LANES,), jnp.float32),
            pltpu.SemaphoreType.DMA,
            pltpu.SemaphoreType.DMA,
        ],
        compiler_params=pltpu.CompilerParams(needs_layout_passes=False),
    )
    parts = run(features_t, targets, centers_t)
    return jnp.sum(parts)


def kernel(features, targets, centers):
    return _center_loss(features.T, targets.astype(jnp.int32), centers.T)
